# baseline (speedup 1.0000x reference)
import functools

import jax
import jax.numpy as jnp
from jax import lax
from jax.experimental import pallas as pl
from jax.experimental.pallas import tpu as pltpu
from jax.experimental.pallas import tpu_sc as plsc

_NC = 2
_NS = 16
_NW = _NC * _NS

_CH = 128
_NBUF = 2


@functools.cache
def _build(B, V, D):
    assert B % (_NW * _CH) == 0
    bpw = B // _NW
    nch = bpw // _CH

    mesh = plsc.VectorSubcoreMesh(core_axis_name="c", subcore_axis_name="s")

    @functools.partial(
        pl.kernel,
        mesh=mesh,
        out_type=jax.ShapeDtypeStruct((B, D), jnp.float32),
        scratch_types=[
            pltpu.VMEM((bpw,), jnp.int32),
            pltpu.VMEM((_NBUF, _CH, D), jnp.float32),
            pltpu.SemaphoreType.DMA,
        ],
        compiler_params=pltpu.CompilerParams(use_tc_tiling_on_sc=False),
    )
    def gather_kernel(table_hbm, idx_hbm, out_hbm, idx_v, rows_v, gsem):
        wid = lax.axis_index("s") * _NC + lax.axis_index("c")
        base = wid * bpw
        pltpu.sync_copy(idx_hbm.at[pl.ds(base, bpw)], idx_v)

        for b in range(_NBUF):
            pltpu.make_async_copy(
                table_hbm.at[idx_v.at[pl.ds(b * _CH, _CH)]],
                rows_v.at[b],
                gsem,
            ).start()

        def body(j, carry):
            slot = lax.rem(j, _NBUF)
            pltpu.make_async_copy(
                table_hbm.at[idx_v.at[pl.ds(j * _CH, _CH)]],
                rows_v.at[slot],
                gsem,
            ).wait()
            pltpu.sync_copy(
                rows_v.at[slot],
                out_hbm.at[pl.ds(base + j * _CH, _CH)],
            )

            @pl.when(j + _NBUF < nch)
            def _():
                pltpu.make_async_copy(
                    table_hbm.at[idx_v.at[pl.ds((j + _NBUF) * _CH, _CH)]],
                    rows_v.at[slot],
                    gsem,
                ).start()

            return carry

        lax.fori_loop(0, nch, body, 0)

    return gather_kernel


def kernel(questions, table):
    Bq, S = questions.shape
    V, D = table.shape
    idx = questions.reshape(-1).astype(jnp.int32)
    out = _build(Bq * S, V, D)(table, idx)
    return out.reshape(Bq, S, D)


# speedup vs baseline: 1.0197x; 1.0197x over previous
import functools

import jax
import jax.numpy as jnp
from jax import lax
from jax.experimental import pallas as pl
from jax.experimental.pallas import tpu as pltpu
from jax.experimental.pallas import tpu_sc as plsc

_NC = 2
_NS = 16
_NW = _NC * _NS

_CH = 128
_K = 4


@functools.cache
def _build(B, V, D):
    assert B % (_NW * _CH * _K) == 0
    bpw = B // _NW
    nch = bpw // _CH
    nss = nch // _K

    mesh = plsc.VectorSubcoreMesh(core_axis_name="c", subcore_axis_name="s")

    @functools.partial(
        pl.kernel,
        mesh=mesh,
        out_type=jax.ShapeDtypeStruct((B, D), jnp.float32),
        scratch_types=[
            pltpu.VMEM((bpw,), jnp.int32),
            pltpu.VMEM((2, _K, _CH, D), jnp.float32),
            pltpu.SemaphoreType.DMA,
            pltpu.SemaphoreType.DMA,
        ],
        compiler_params=pltpu.CompilerParams(use_tc_tiling_on_sc=False),
    )
    def gather_kernel(table_hbm, idx_hbm, out_hbm, idx_v, rows_v, gsem, wsem):
        wid = lax.axis_index("s") * _NC + lax.axis_index("c")
        base = wid * bpw
        pltpu.sync_copy(idx_hbm.at[pl.ds(base, bpw)], idx_v)

        def fire_gathers(s, g):
            for b in range(_K):
                pltpu.make_async_copy(
                    table_hbm.at[idx_v.at[pl.ds((s * _K + b) * _CH, _CH)]],
                    rows_v.at[g, b],
                    gsem,
                ).start()

        fire_gathers(0, 0)

        def body(s, carry):
            g = lax.rem(s, 2)

            @pl.when(s + 1 < nss)
            def _():
                fire_gathers(s + 1, lax.rem(s + 1, 2))

            for b in range(_K):
                pltpu.make_async_copy(
                    table_hbm.at[idx_v.at[pl.ds((s * _K + b) * _CH, _CH)]],
                    rows_v.at[g, b],
                    gsem,
                ).wait()
            for b in range(_K):
                pltpu.make_async_copy(
                    rows_v.at[g, b],
                    out_hbm.at[pl.ds(base + (s * _K + b) * _CH, _CH)],
                    wsem,
                ).start()
            for b in range(_K):
                pltpu.make_async_copy(
                    rows_v.at[g, b],
                    out_hbm.at[pl.ds(base + (s * _K + b) * _CH, _CH)],
                    wsem,
                ).wait()
            return carry

        lax.fori_loop(0, nss, body, 0)

    return gather_kernel


def kernel(questions, table):
    Bq, S = questions.shape
    V, D = table.shape
    idx = questions.reshape(-1).astype(jnp.int32)
    out = _build(Bq * S, V, D)(table, idx)
    return out.reshape(Bq, S, D)


# speedup vs baseline: 1.0435x; 1.0234x over previous
import functools

import jax
import jax.numpy as jnp
from jax import lax
from jax.experimental import pallas as pl
from jax.experimental.pallas import tpu as pltpu
from jax.experimental.pallas import tpu_sc as plsc

_NC = 2
_NS = 16
_NW = _NC * _NS

_CH = 128
_K = 4


@functools.cache
def _build(B, V, D):
    assert B % (_NW * _CH * _K) == 0
    bpw = B // _NW
    nch = bpw // _CH
    nss = nch // _K

    mesh = plsc.VectorSubcoreMesh(core_axis_name="c", subcore_axis_name="s")

    @functools.partial(
        pl.kernel,
        mesh=mesh,
        out_type=jax.ShapeDtypeStruct((B, D), jnp.float32),
        scratch_types=[
            pltpu.VMEM((bpw,), jnp.int32),
            pltpu.VMEM((2, _K, _CH, D), jnp.float32),
            pltpu.SemaphoreType.DMA,
            pltpu.SemaphoreType.DMA,
        ],
        compiler_params=pltpu.CompilerParams(use_tc_tiling_on_sc=False),
    )
    def gather_kernel(table_hbm, idx_hbm, out_hbm, idx_v, rows_v, gsem, wsem):
        wid = lax.axis_index("s") * _NC + lax.axis_index("c")
        base = wid * bpw
        pltpu.sync_copy(idx_hbm.at[pl.ds(base, bpw)], idx_v)

        def fire_gathers(s, g):
            for b in range(_K):
                pltpu.make_async_copy(
                    table_hbm.at[idx_v.at[pl.ds((s * _K + b) * _CH, _CH)]],
                    rows_v.at[g, b],
                    gsem,
                ).start()

        fire_gathers(0, 0)

        def body(s, carry):
            g = lax.rem(s, 2)

            @pl.when(s + 1 < nss)
            def _():
                fire_gathers(s + 1, lax.rem(s + 1, 2))

            for b in range(_K):
                pltpu.make_async_copy(
                    table_hbm.at[idx_v.at[pl.ds((s * _K + b) * _CH, _CH)]],
                    rows_v.at[g, b],
                    gsem,
                ).wait()
            for b in range(_K):
                pltpu.make_async_copy(
                    rows_v.at[g, b],
                    out_hbm.at[pl.ds(base + (s * _K + b) * _CH, _CH)],
                    wsem,
                ).start()
            for b in range(_K):
                pltpu.make_async_copy(
                    rows_v.at[g, b],
                    out_hbm.at[pl.ds(base + (s * _K + b) * _CH, _CH)],
                    wsem,
                ).wait()
            return carry

        lax.fori_loop(0, nss, body, 0)

    return gather_kernel


def kernel(questions, table):
    Bq, S = questions.shape
    V, D = table.shape
    idx = questions.T.reshape(-1).astype(jnp.int32)
    out = _build(Bq * S, V, D)(table, idx)
    return out.reshape(S, Bq, D).transpose(1, 0, 2)
